# baseline (speedup 1.0000x reference)
import functools

import jax
import jax.numpy as jnp
from jax import lax
from jax.experimental import pallas as pl
from jax.experimental.pallas import tpu as pltpu
from jax.experimental.pallas import tpu_sc as plsc

NC = 2
NS = 16
NW = NC * NS

B = 16384
D = 32
D_SCAL = 16

B_PER_W = B // NW
CHUNK = 128
NCHUNK = B_PER_W // CHUNK


def _make_sc_gather():
    mesh = plsc.VectorSubcoreMesh(core_axis_name="c", subcore_axis_name="s")

    @functools.partial(
        pl.kernel,
        mesh=mesh,
        out_type=jax.ShapeDtypeStruct((B, D), jnp.float32),
        compiler_params=pltpu.CompilerParams(use_tc_tiling_on_sc=False),
        scratch_types=[
            pltpu.VMEM((NCHUNK, CHUNK), jnp.int32),
            pltpu.VMEM((B_PER_W, D), jnp.float32),
            pltpu.SemaphoreType.DMA,
        ],
    )
    def gather_kernel(table_hbm, idx_hbm, out_hbm, idx_v, rows_v, sem):
        wid = lax.axis_index("s") * NC + lax.axis_index("c")
        base = wid * B_PER_W
        pltpu.sync_copy(idx_hbm.at[wid], idx_v)
        copies = []
        for j in range(NCHUNK):
            copies.append(
                pltpu.async_copy(
                    table_hbm.at[idx_v.at[j]],
                    rows_v.at[pl.ds(j * CHUNK, CHUNK)],
                    sem,
                )
            )
        for c in copies:
            c.wait()
        pltpu.sync_copy(rows_v, out_hbm.at[pl.ds(base, B_PER_W)])

    return gather_kernel


_sc_gather = _make_sc_gather()


def _mm_body(x_ref, w_ref, b_ref, o_ref):
    o_ref[...] = (
        jax.lax.dot_general(
            x_ref[...], w_ref[...],
            dimension_numbers=(((1,), (0,)), ((), ())),
            preferred_element_type=jnp.float32,
            precision=jax.lax.Precision.HIGHEST,
        )
        + b_ref[...]
    )


def _tc_matmul(scal_feat, W_scal, b_scal):
    BLK = 4096
    return pl.pallas_call(
        _mm_body,
        grid=(B // BLK,),
        in_specs=[
            pl.BlockSpec((BLK, D_SCAL), lambda i: (i, 0)),
            pl.BlockSpec((D_SCAL, D), lambda i: (0, 0)),
            pl.BlockSpec((1, D), lambda i: (0, 0)),
        ],
        out_specs=pl.BlockSpec((BLK, D), lambda i: (i, 0)),
        out_shape=jax.ShapeDtypeStruct((B, D), jnp.float32),
    )(scal_feat, W_scal, b_scal.reshape(1, D))


def kernel(scal_feat, cat_feat, W_scal, b_scal, emb_table):
    idx = cat_feat.astype(jnp.int32).reshape(NW, NCHUNK, CHUNK)
    cat = _sc_gather(emb_table, idx)
    scal = _tc_matmul(scal_feat, W_scal, b_scal)
    return jnp.concatenate((scal, cat), axis=-1)


# speedup vs baseline: 3.6180x; 3.6180x over previous
import functools

import jax
import jax.numpy as jnp
from jax import lax
from jax.experimental import pallas as pl
from jax.experimental.pallas import tpu as pltpu
from jax.experimental.pallas import tpu_sc as plsc

NC = 2
NS = 16
NW = NC * NS

B = 16384
D = 32
D_SCAL = 16

B_PER_W = B // NW
CH = 16
NCHUNK = B_PER_W // CH
W_SLAB = 128


def _make_sc_gather():
    mesh = plsc.VectorSubcoreMesh(core_axis_name="c", subcore_axis_name="s")

    @functools.partial(
        pl.kernel,
        mesh=mesh,
        out_type=jax.ShapeDtypeStruct((D, B), jnp.float32),
        compiler_params=pltpu.CompilerParams(needs_layout_passes=False),
        scratch_types=[
            pltpu.VMEM((B_PER_W,), jnp.int32),
            pltpu.VMEM((CH, D, W_SLAB), jnp.float32),
            pltpu.VMEM((D, B_PER_W), jnp.float32),
            pltpu.SemaphoreType.DMA,
        ],
    )
    def gather_kernel(table_t, idx_hbm, out_t, idx_v, slab, stage, sem):
        wid = lax.axis_index("s") * NC + lax.axis_index("c")
        base = wid * B_PER_W
        pltpu.sync_copy(idx_hbm.at[pl.ds(base, B_PER_W)], idx_v)

        rows_lo = lax.iota(jnp.int32, 16)
        rows_hi = rows_lo + 16
        lane16 = lax.iota(jnp.int32, 16)

        @pl.loop(0, NCHUNK)
        def _(c):
            j0 = c * CH
            i_vec = idx_v[pl.ds(j0, CH)]
            scal_idx = []
            for r in range(CH):
                sel = jnp.where(lane16 == r, i_vec, jnp.zeros_like(i_vec))
                scal_idx.append(lax.reduce_max(sel, axes=(0,)))
            copies = []
            for r in range(CH):
                off = pl.multiple_of(
                    (scal_idx[r] // W_SLAB) * W_SLAB, 128
                )
                copies.append(
                    pltpu.async_copy(
                        table_t.at[:, pl.ds(off, W_SLAB)], slab.at[r], sem
                    )
                )
            for c_ in copies:
                c_.wait()
            for r in range(CH):
                lane = jnp.full((16,), scal_idx[r] % W_SLAB, jnp.int32)
                cols = jnp.full((16,), j0 + r, jnp.int32)
                v_lo = plsc.load_gather(slab.at[r], [rows_lo, lane])
                v_hi = plsc.load_gather(slab.at[r], [rows_hi, lane])
                plsc.store_scatter(stage, [rows_lo, cols], v_lo)
                plsc.store_scatter(stage, [rows_hi, cols], v_hi)

        pltpu.sync_copy(stage, out_t.at[:, pl.ds(base, B_PER_W)])

    return gather_kernel


_sc_gather = _make_sc_gather()


def _mm_body(w_ref, x_ref, b_ref, o_ref):
    o_ref[...] = (
        jax.lax.dot_general(
            w_ref[...], x_ref[...],
            dimension_numbers=(((0,), (0,)), ((), ())),
            preferred_element_type=jnp.float32,
            precision=jax.lax.Precision.HIGHEST,
        )
        + b_ref[...]
    )


def _tc_matmul_t(W_scal, scal_t, b_col):
    BLK = 2048
    return pl.pallas_call(
        _mm_body,
        grid=(B // BLK,),
        in_specs=[
            pl.BlockSpec((D_SCAL, D), lambda i: (0, 0)),
            pl.BlockSpec((D_SCAL, BLK), lambda i: (0, i)),
            pl.BlockSpec((D, 1), lambda i: (0, 0)),
        ],
        out_specs=pl.BlockSpec((D, BLK), lambda i: (0, i)),
        out_shape=jax.ShapeDtypeStruct((D, B), jnp.float32),
    )(W_scal, scal_t, b_col)


def kernel(scal_feat, cat_feat, W_scal, b_scal, emb_table):
    idx = cat_feat.astype(jnp.int32)
    cat_t = _sc_gather(emb_table.T, idx)
    scal_t = _tc_matmul_t(W_scal, scal_feat.T, b_scal.reshape(D, 1))
    return jnp.concatenate((scal_t, cat_t), axis=0).T


# speedup vs baseline: 3.6457x; 1.0077x over previous
import functools

import jax
import jax.numpy as jnp
from jax import lax
from jax.experimental import pallas as pl
from jax.experimental.pallas import tpu as pltpu
from jax.experimental.pallas import tpu_sc as plsc

NC = 2
NS = 16
NW = NC * NS

B = 16384
D = 32
D_SCAL = 16

B_PER_W = B // NW
CH = 16
NCHUNK = B_PER_W // CH
W_SLAB = 128


def _make_sc_gather():
    mesh = plsc.VectorSubcoreMesh(core_axis_name="c", subcore_axis_name="s")

    @functools.partial(
        pl.kernel,
        mesh=mesh,
        out_type=jax.ShapeDtypeStruct((2 * D, B), jnp.float32),
        compiler_params=pltpu.CompilerParams(needs_layout_passes=False),
        scratch_types=[
            pltpu.VMEM((B_PER_W,), jnp.int32),
            pltpu.VMEM((CH, D, W_SLAB), jnp.float32),
            pltpu.VMEM((D, B_PER_W), jnp.float32),
            pltpu.SemaphoreType.DMA,
        ],
    )
    def gather_kernel(table_t, idx_hbm, scal_t, out_t, idx_v, slab, stage, sem):
        wid = lax.axis_index("s") * NC + lax.axis_index("c")
        base = wid * B_PER_W
        pltpu.sync_copy(idx_hbm.at[pl.ds(base, B_PER_W)], idx_v)
        scal_copy = pltpu.make_async_copy(
            scal_t.at[:, pl.ds(base, B_PER_W)],
            out_t.at[pl.ds(0, D), pl.ds(base, B_PER_W)],
            sem,
        )
        scal_copy.start()

        rows_lo = lax.iota(jnp.int32, 16)
        rows_hi = rows_lo + 16
        lane16 = lax.iota(jnp.int32, 16)

        @pl.loop(0, NCHUNK)
        def _(c):
            j0 = c * CH
            i_vec = idx_v[pl.ds(j0, CH)]
            scal_idx = []
            for r in range(CH):
                sel = jnp.where(lane16 == r, i_vec, jnp.zeros_like(i_vec))
                scal_idx.append(lax.reduce_max(sel, axes=(0,)))
            copies = []
            for r in range(CH):
                off = pl.multiple_of(
                    (scal_idx[r] // W_SLAB) * W_SLAB, 128
                )
                copies.append(
                    pltpu.async_copy(
                        table_t.at[:, pl.ds(off, W_SLAB)], slab.at[r], sem
                    )
                )
            for c_ in copies:
                c_.wait()
            for r in range(CH):
                lane = jnp.full((16,), scal_idx[r] % W_SLAB, jnp.int32)
                cols = jnp.full((16,), j0 + r, jnp.int32)
                v_lo = plsc.load_gather(slab.at[r], [rows_lo, lane])
                v_hi = plsc.load_gather(slab.at[r], [rows_hi, lane])
                plsc.store_scatter(stage, [rows_lo, cols], v_lo)
                plsc.store_scatter(stage, [rows_hi, cols], v_hi)

        scal_copy.wait()
        pltpu.sync_copy(stage, out_t.at[pl.ds(D, D), pl.ds(base, B_PER_W)])

    return gather_kernel


_sc_gather = _make_sc_gather()


def _mm_body(w_ref, x_ref, b_ref, o_ref):
    o_ref[...] = (
        jax.lax.dot_general(
            w_ref[...], x_ref[...],
            dimension_numbers=(((0,), (0,)), ((), ())),
            preferred_element_type=jnp.float32,
            precision=jax.lax.Precision.HIGHEST,
        )
        + b_ref[...]
    )


def _tc_matmul_t(W_scal, scal_t, b_col):
    BLK = 2048
    return pl.pallas_call(
        _mm_body,
        grid=(B // BLK,),
        in_specs=[
            pl.BlockSpec((D_SCAL, D), lambda i: (0, 0)),
            pl.BlockSpec((D_SCAL, BLK), lambda i: (0, i)),
            pl.BlockSpec((D, 1), lambda i: (0, 0)),
        ],
        out_specs=pl.BlockSpec((D, BLK), lambda i: (0, i)),
        out_shape=jax.ShapeDtypeStruct((D, B), jnp.float32),
    )(W_scal, scal_t, b_col)


def kernel(scal_feat, cat_feat, W_scal, b_scal, emb_table):
    idx = cat_feat.astype(jnp.int32)
    scal_t = _tc_matmul_t(W_scal, scal_feat.T, b_scal.reshape(D, 1))
    out_t = _sc_gather(emb_table.T, idx, scal_t)
    return out_t.T
